# baseline (speedup 1.0000x reference)
import functools

import jax
import jax.numpy as jnp
from jax import lax
from jax.experimental import pallas as pl
from jax.experimental.pallas import tpu as pltpu
from jax.experimental.pallas import tpu_sc as plsc

_NUM_CORES = 2
_NUM_SUBCORES = 16
_NW = _NUM_CORES * _NUM_SUBCORES

_LANE = 128
_K = 8


def _build(n_total, vocab, dim):
  n_per_w = n_total // _NW
  chunk = _K * _LANE
  n_steps = n_per_w // chunk
  mesh = plsc.VectorSubcoreMesh(core_axis_name="c", subcore_axis_name="s")

  @functools.partial(
      pl.kernel,
      out_type=jax.ShapeDtypeStruct((n_total, dim), jnp.float32),
      mesh=mesh,
      scratch_types=[
          pltpu.VMEM((_K, _LANE), jnp.int32),
          pltpu.VMEM((_K, _LANE, dim), jnp.float32),
          pltpu.SemaphoreType.DMA,
          pltpu.SemaphoreType.DMA,
      ],
      compiler_params=pltpu.CompilerParams(use_tc_tiling_on_sc=False),
  )
  def lookup(ids_hbm, table_hbm, out_hbm, idx_v, rows_v, gsem, osem):
    wid = lax.axis_index("s") * _NUM_CORES + lax.axis_index("c")
    base = wid * n_per_w

    def body(step, carry):
      off = pl.multiple_of(base + step * chunk, chunk)
      row = pl.multiple_of(wid * (n_per_w // _LANE) + step * _K, _K)
      pltpu.sync_copy(ids_hbm.at[pl.ds(row, _K)], idx_v)
      for j in range(_K):
        pltpu.async_copy(table_hbm.at[idx_v.at[j]], rows_v.at[j], gsem)
      for j in range(_K):
        pltpu.make_async_copy(table_hbm.at[idx_v.at[j]], rows_v.at[j],
                              gsem).wait()
      for j in range(_K):
        pltpu.async_copy(rows_v.at[j],
                         out_hbm.at[pl.ds(pl.multiple_of(off + j * _LANE,
                                                         _LANE), _LANE)],
                         osem)
      for j in range(_K):
        pltpu.make_async_copy(rows_v.at[j],
                              out_hbm.at[pl.ds(pl.multiple_of(
                                  off + j * _LANE, _LANE), _LANE)],
                              osem).wait()
      return carry

    lax.fori_loop(0, n_steps, body, 0)

  return lookup


def kernel(token_ids, W):
  b, s = token_ids.shape
  vocab, dim = W.shape
  n_total = b * s
  ids = token_ids.reshape(n_total // _LANE, _LANE).astype(jnp.int32)
  out = _build(n_total, vocab, dim)(ids, W)
  return out.reshape(b, s, dim)
